# Initial kernel scaffold; baseline (speedup 1.0000x reference)
import jax
import jax.numpy as jnp
from jax.experimental import pallas as pl


def kernel(x, edge_index, cluster_id, cluster_index, W1, att_src1, att_dst1, b1, W2, att_src2, att_dst2, b2):
    raise NotImplementedError("write your pallas kernel here")


# speedup vs baseline: 1.2340x; 1.2340x over previous
import jax
import jax.numpy as jnp
from jax.experimental import pallas as pl

N = 10000
E = 320000
NFEAT = 128
NHID = 16
HEADS = 8
NCLASS = 10
CLUSTER = 20
NTRAIN = 5000


def _mm_kernel(x_ref, w_ref, o_ref):
    o_ref[...] = jnp.dot(x_ref[...], w_ref[...], preferred_element_type=jnp.float32)


def _matmul(x, w):
    m, k = x.shape
    _, n = w.shape
    bm = 1000
    return pl.pallas_call(
        _mm_kernel,
        grid=(m // bm,),
        in_specs=[
            pl.BlockSpec((bm, k), lambda i: (i, 0)),
            pl.BlockSpec((k, n), lambda i: (0, 0)),
        ],
        out_specs=pl.BlockSpec((bm, n), lambda i: (i, 0)),
        out_shape=jax.ShapeDtypeStruct((m, n), jnp.float32),
    )(x, w)


def _gat_conv(x, edge_index, W, att_src, att_dst, bias, heads, out_ch, concat, num_nodes):
    h = _matmul(x, W).reshape(num_nodes, heads, out_ch)
    loop = jnp.arange(num_nodes)
    src = jnp.concatenate([edge_index[0], loop])
    dst = jnp.concatenate([edge_index[1], loop])
    a_src = jnp.sum(h * att_src, axis=-1)
    a_dst = jnp.sum(h * att_dst, axis=-1)
    alpha = jax.nn.leaky_relu(a_src[src] + a_dst[dst], negative_slope=0.2)
    ex = jnp.exp(alpha)
    denom = jax.ops.segment_sum(ex, dst, num_segments=num_nodes)
    num = jax.ops.segment_sum(h[src] * ex[:, :, None], dst, num_segments=num_nodes)
    out = num / (denom[:, :, None] + 1e-16)
    if concat:
        out = out.reshape(num_nodes, heads * out_ch)
    else:
        out = out.mean(axis=1)
    return out + bias


def kernel(x, edge_index, cluster_id, cluster_index, W1, att_src1, att_dst1, b1,
           W2, att_src2, att_dst2, b2):
    h = _gat_conv(x, edge_index, W1, att_src1, att_dst1, b1, HEADS, NHID, True, N)
    h = jax.nn.elu(h)
    sel_id = cluster_id[cluster_index]
    sel_x = h[cluster_index]
    cluster_features = (sel_id.T @ sel_x) / sel_id.sum(0)[:, None]
    x1 = cluster_features[jnp.argmax(cluster_id, axis=1)]
    xcat = jnp.concatenate([jnp.concatenate([h, x1], axis=1),
                            jnp.concatenate([x1, h], axis=1)], axis=0)
    ei2 = jnp.concatenate([edge_index, edge_index + N], axis=1)
    out = _gat_conv(xcat, ei2, W2, att_src2, att_dst2, b2, 1, NCLASS * NCLASS, False, 2 * N)
    return out


# speedup vs baseline: 32.1831x; 26.0793x over previous
import functools

import jax
import jax.numpy as jnp
from jax import lax
from jax.experimental import pallas as pl
from jax.experimental.pallas import tpu as pltpu
from jax.experimental.pallas import tpu_sc as plsc

N = 10000
E = 320000
NFEAT = 128
NHID = 16
HEADS = 8
NCLASS = 10
CLUSTER = 20
NTRAIN = 5000

_C = 80
_D1 = 144
_D2 = 112
_NTILE = 16
_RPT = 624
_TAIL = N - _NTILE * _RPT
_LANE = 16
_SC_PARAMS = pltpu.CompilerParams(use_tc_tiling_on_sc=False)


def _mm_kernel(x_ref, w_ref, o_ref):
    o_ref[...] = jnp.dot(x_ref[...], w_ref[...], preferred_element_type=jnp.float32)


def _matmul(x, w):
    m, k = x.shape
    _, n = w.shape
    bm = 1000
    return pl.pallas_call(
        _mm_kernel,
        grid=(m // bm,),
        in_specs=[
            pl.BlockSpec((bm, k), lambda i: (i, 0)),
            pl.BlockSpec((k, n), lambda i: (0, 0)),
        ],
        out_specs=pl.BlockSpec((bm, n), lambda i: (i, 0)),
        out_shape=jax.ShapeDtypeStruct((m, n), jnp.float32),
    )(x, w)


def _splat(v, lane):
    idx = jnp.full((_LANE, 1), lane, dtype=jnp.int32)
    return lax.gather(
        v, idx,
        lax.GatherDimensionNumbers(offset_dims=(), collapsed_slice_dims=(0,),
                                   start_index_map=(0,)),
        slice_sizes=(1,), mode=lax.GatherScatterMode.PROMISE_IN_BOUNDS)


def _zero_tile_slice(stage, acc, s, d):
    zv = jnp.zeros((_LANE,), jnp.float32)

    def zrow(i, _):
        for hh in range(d // _LANE):
            stage[i, pl.ds(_LANE * hh, _LANE)] = zv
        return 0

    lax.fori_loop(0, _C, zrow, 0)
    r0 = s * _RPT
    nfull = _RPT // _C
    rem = _RPT - nfull * _C

    def zcp(j, _):
        pltpu.sync_copy(stage, acc.at[pl.ds(r0 + j * _C, _C)])
        return 0

    lax.fori_loop(0, nfull, zcp, 0)
    if rem:
        pltpu.sync_copy(stage.at[pl.ds(0, rem)], acc.at[pl.ds(r0 + nfull * _C, rem)])

    @pl.when(s == _NTILE - 1)
    def _():
        pltpu.sync_copy(stage.at[pl.ds(0, _TAIL)],
                        acc.at[pl.ds(_NTILE * _RPT, _TAIL)])


def _copy_out_tile(acc, out_h, c, s):
    r0 = s * _RPT
    pltpu.sync_copy(acc.at[pl.ds(r0, _RPT)], out_h.at[c, pl.ds(r0, _RPT)])

    @pl.when(s == _NTILE - 1)
    def _():
        pltpu.sync_copy(acc.at[pl.ds(_NTILE * _RPT, _TAIL)],
                        out_h.at[c, pl.ds(_NTILE * _RPT, _TAIL)])


def _sc_gat1(t1, adst, src, dst):
    nw_edges = E // 32
    nchunk = nw_edges // _C
    mesh = plsc.VectorSubcoreMesh(core_axis_name="c", subcore_axis_name="s")

    @functools.partial(
        pl.kernel, mesh=mesh, compiler_params=_SC_PARAMS,
        out_type=jax.ShapeDtypeStruct((2, N, _D1), jnp.float32),
        scratch_types=[
            pltpu.VMEM_SHARED((N, _D1), jnp.float32),
            pltpu.VMEM((_C,), jnp.int32),
            pltpu.VMEM((_C,), jnp.int32),
            pltpu.VMEM((_C, _D1), jnp.float32),
            pltpu.VMEM((_C, _LANE), jnp.float32),
            pltpu.VMEM((_C, _D1), jnp.float32),
            pltpu.SemaphoreType.DMA,
        ],
    )
    def k(t1_h, adst_h, src_h, dst_h, out_h, acc, sidx, didx, srows, arows,
          stage, sem):
        c = lax.axis_index("c")
        s = lax.axis_index("s")
        _zero_tile_slice(stage, acc, s, _D1)
        plsc.subcore_barrier()

        base = c * (E // 2) + s * nw_edges

        def chunk(i, _):
            off = base + i * _C
            pltpu.sync_copy(src_h.at[pl.ds(off, _C)], sidx)
            pltpu.sync_copy(dst_h.at[pl.ds(off, _C)], didx)
            pltpu.async_copy(t1_h.at[sidx], srows, sem).wait()
            pltpu.async_copy(adst_h.at[didx], arows, sem).wait()

            def edge(e, _):
                av = srows[e, pl.ds(128, _LANE)]
                bv = arows[e, pl.ds(0, _LANE)]
                t = av + bv
                t = jnp.where(t >= 0.0, t, 0.2 * t)
                w = jnp.exp(t)
                for hh in range(HEADS):
                    sp = _splat(w, hh)
                    stage[e, pl.ds(_LANE * hh, _LANE)] = (
                        sp * srows[e, pl.ds(_LANE * hh, _LANE)])
                stage[e, pl.ds(128, _LANE)] = w
                return 0

            lax.fori_loop(0, _C, edge, 0)
            pltpu.sync_copy(stage, acc.at[didx], add=True)
            return 0

        lax.fori_loop(0, nchunk, chunk, 0)
        plsc.subcore_barrier()
        _copy_out_tile(acc, out_h, c, s)

    return k(t1, adst, src, dst)


def _sc_gat2(t2, adst2, src, dst):
    nw_edges = E // _NTILE
    nchunk = nw_edges // _C
    mesh = plsc.VectorSubcoreMesh(core_axis_name="c", subcore_axis_name="s")

    @functools.partial(
        pl.kernel, mesh=mesh, compiler_params=_SC_PARAMS,
        out_type=jax.ShapeDtypeStruct((2, N, _D2), jnp.float32),
        scratch_types=[
            pltpu.VMEM_SHARED((N, _D2), jnp.float32),
            pltpu.VMEM((_C,), jnp.int32),
            pltpu.VMEM((_C,), jnp.int32),
            pltpu.VMEM((_C,), jnp.int32),
            pltpu.VMEM((_C, _D2), jnp.float32),
            pltpu.VMEM((_C, _LANE), jnp.float32),
            pltpu.VMEM((_C, _D2), jnp.float32),
            pltpu.SemaphoreType.DMA,
        ],
    )
    def k(t2_h, adst_h, src_h, dst_h, out_h, acc, sidx, didx, gidx, srows,
          arows, stage, sem):
        c = lax.axis_index("c")
        s = lax.axis_index("s")
        _zero_tile_slice(stage, acc, s, _D2)
        plsc.subcore_barrier()

        base = s * nw_edges
        voff = c * N
        lane4 = jnp.arange(_LANE, dtype=jnp.int32) == 4

        def chunk(i, _):
            off = base + i * _C
            pltpu.sync_copy(src_h.at[pl.ds(off, _C)], sidx)
            pltpu.sync_copy(dst_h.at[pl.ds(off, _C)], didx)
            def shift(v, _2):
                gidx[pl.ds(_LANE * v, _LANE)] = (
                    sidx[pl.ds(_LANE * v, _LANE)] + voff)
                return 0
            lax.fori_loop(0, _C // _LANE, shift, 0)
            pltpu.async_copy(t2_h.at[gidx], srows, sem).wait()

            def shiftd(v, _2):
                gidx[pl.ds(_LANE * v, _LANE)] = (
                    didx[pl.ds(_LANE * v, _LANE)] + voff)
                return 0
            lax.fori_loop(0, _C // _LANE, shiftd, 0)
            pltpu.async_copy(adst_h.at[gidx], arows, sem).wait()

            def edge(e, _):
                rv6 = srows[e, pl.ds(96, _LANE)]
                t = rv6 + arows[e, pl.ds(0, _LANE)]
                t = jnp.where(t >= 0.0, t, 0.2 * t)
                w = jnp.exp(t)
                sp = _splat(w, 4)
                for kk in range(7):
                    sv = sp * srows[e, pl.ds(_LANE * kk, _LANE)]
                    if kk == 6:
                        sv = jnp.where(lane4, sp, sv)
                    stage[e, pl.ds(_LANE * kk, _LANE)] = sv
                return 0

            lax.fori_loop(0, _C, edge, 0)
            pltpu.sync_copy(stage, acc.at[didx], add=True)
            return 0

        lax.fori_loop(0, nchunk, chunk, 0)
        plsc.subcore_barrier()
        _copy_out_tile(acc, out_h, c, s)

    return k(t2, adst2, src, dst)


def kernel(x, edge_index, cluster_id, cluster_index, W1, att_src1, att_dst1, b1,
           W2, att_src2, att_dst2, b2):
    src = edge_index[0].astype(jnp.int32)
    dst = edge_index[1].astype(jnp.int32)

    h = _matmul(x, W1)
    hr = h.reshape(N, HEADS, NHID)
    a_src = jnp.sum(hr * att_src1, axis=-1)
    a_dst = jnp.sum(hr * att_dst1, axis=-1)
    zpad = jnp.zeros((N, 8), jnp.float32)
    t1 = jnp.concatenate([h, a_src, zpad], axis=1)
    adst_tab = jnp.concatenate([a_dst, zpad], axis=1)

    acc = _sc_gat1(t1, adst_tab, src, dst)
    accs = acc[0] + acc[1]
    w_self = jnp.exp(jax.nn.leaky_relu(a_src + a_dst, negative_slope=0.2))
    num = accs[:, :128].reshape(N, HEADS, NHID) + w_self[:, :, None] * hr
    den = accs[:, 128:136] + w_self
    h1 = (num / (den[:, :, None] + 1e-16)).reshape(N, HEADS * NHID) + b1
    h1 = jax.nn.elu(h1)

    sel_id = cluster_id[cluster_index]
    sel_x = h1[cluster_index]
    cf = (sel_id.T @ sel_x) / sel_id.sum(0)[:, None]
    cidx = jnp.argmax(cluster_id, axis=1)

    W2a, W2b = W2[:128], W2[128:]
    hA = _matmul(h1, W2a)
    hB = _matmul(h1, W2b)
    cfA = cf @ W2a
    cfB = cf @ W2b
    h2lo = hA + cfB[cidx]
    h2hi = hB + cfA[cidx]
    vs2 = att_src2.reshape(NCLASS * NCLASS)
    vd2 = att_dst2.reshape(NCLASS * NCLASS)
    as_lo = h2lo @ vs2
    as_hi = h2hi @ vs2
    ad_lo = h2lo @ vd2
    ad_hi = h2hi @ vd2

    zpad2 = jnp.zeros((N, 11), jnp.float32)
    t2 = jnp.concatenate([
        jnp.concatenate([h2lo, as_lo[:, None], zpad2], axis=1),
        jnp.concatenate([h2hi, as_hi[:, None], zpad2], axis=1)], axis=0)
    adst2_tab = jnp.concatenate([
        jnp.broadcast_to(ad_lo[:, None], (N, 16)),
        jnp.broadcast_to(ad_hi[:, None], (N, 16))], axis=0)

    acc2 = _sc_gat2(t2, adst2_tab, src, dst)

    def finish(acch, h2, a_s, a_d):
        ws = jnp.exp(jax.nn.leaky_relu(a_s + a_d, negative_slope=0.2))
        numv = acch[:, :100] + ws[:, None] * h2
        denv = acch[:, 100:101] + ws[:, None]
        return numv / (denv + 1e-16)

    out_lo = finish(acc2[0], h2lo, as_lo, ad_lo)
    out_hi = finish(acc2[1], h2hi, as_hi, ad_hi)
    return jnp.concatenate([out_lo, out_hi], axis=0) + b2


# speedup vs baseline: 47.1587x; 1.4653x over previous
import functools

import jax
import jax.numpy as jnp
from jax import lax
from jax.experimental import pallas as pl
from jax.experimental.pallas import tpu as pltpu
from jax.experimental.pallas import tpu_sc as plsc

N = 10000
E = 320000
NFEAT = 128
NHID = 16
HEADS = 8
NCLASS = 10
CLUSTER = 20
NTRAIN = 5000

_C = 128
_D1 = 144
_D2 = 112
_NTILE = 16
_RPT = 624
_TAIL = N - _NTILE * _RPT
_LANE = 16
_SC_PARAMS = pltpu.CompilerParams(use_tc_tiling_on_sc=False)


def _mm_kernel(x_ref, w_ref, o_ref):
    o_ref[...] = jnp.dot(x_ref[...], w_ref[...], preferred_element_type=jnp.float32)


def _matmul(x, w):
    m, k = x.shape
    _, n = w.shape
    bm = 1000
    return pl.pallas_call(
        _mm_kernel,
        grid=(m // bm,),
        in_specs=[
            pl.BlockSpec((bm, k), lambda i: (i, 0)),
            pl.BlockSpec((k, n), lambda i: (0, 0)),
        ],
        out_specs=pl.BlockSpec((bm, n), lambda i: (i, 0)),
        out_shape=jax.ShapeDtypeStruct((m, n), jnp.float32),
    )(x, w)


def _splat(v, lane):
    idx = jnp.full((_LANE, 1), lane, dtype=jnp.int32)
    return lax.gather(
        v, idx,
        lax.GatherDimensionNumbers(offset_dims=(), collapsed_slice_dims=(0,),
                                   start_index_map=(0,)),
        slice_sizes=(1,), mode=lax.GatherScatterMode.PROMISE_IN_BOUNDS)


def _zero_tile_slice(stage, acc, s, d):
    zv = jnp.zeros((_LANE,), jnp.float32)

    def zrow(i, _):
        for hh in range(d // _LANE):
            stage[i, pl.ds(_LANE * hh, _LANE)] = zv
        return 0

    lax.fori_loop(0, _C, zrow, 0)
    r0 = s * _RPT
    nfull = _RPT // _C
    rem = _RPT - nfull * _C

    def zcp(j, _):
        pltpu.sync_copy(stage, acc.at[pl.ds(r0 + j * _C, _C)])
        return 0

    lax.fori_loop(0, nfull, zcp, 0)
    if rem:
        pltpu.sync_copy(stage.at[pl.ds(0, rem)], acc.at[pl.ds(r0 + nfull * _C, rem)])

    @pl.when(s == _NTILE - 1)
    def _():
        pltpu.sync_copy(stage.at[pl.ds(0, _TAIL)],
                        acc.at[pl.ds(_NTILE * _RPT, _TAIL)])


def _copy_out_tile(acc, out_h, c, s):
    r0 = s * _RPT
    pltpu.sync_copy(acc.at[pl.ds(r0, _RPT)], out_h.at[c, pl.ds(r0, _RPT)])

    @pl.when(s == _NTILE - 1)
    def _():
        pltpu.sync_copy(acc.at[pl.ds(_NTILE * _RPT, _TAIL)],
                        out_h.at[c, pl.ds(_NTILE * _RPT, _TAIL)])


def _sc_gat1(t1, adst, src, dst):
    nfull = (E // 2 // _NTILE) // _C
    nlast = (E // 2 - 15 * nfull * _C) // _C
    mesh = plsc.VectorSubcoreMesh(core_axis_name="c", subcore_axis_name="s")

    @functools.partial(
        pl.kernel, mesh=mesh, compiler_params=_SC_PARAMS,
        out_type=jax.ShapeDtypeStruct((2, N, _D1), jnp.float32),
        scratch_types=[
            pltpu.VMEM_SHARED((N, _D1), jnp.float32),
            pltpu.VMEM((_C,), jnp.int32),
            pltpu.VMEM((_C,), jnp.int32),
            pltpu.VMEM((_C, _D1), jnp.float32),
            pltpu.VMEM((_C, _LANE), jnp.float32),
            pltpu.VMEM((_C, _D1), jnp.float32),
            pltpu.SemaphoreType.DMA,
        ],
    )
    def k(t1_h, adst_h, src_h, dst_h, out_h, acc, sidx, didx, srows, arows,
          stage, sem):
        c = lax.axis_index("c")
        s = lax.axis_index("s")
        _zero_tile_slice(stage, acc, s, _D1)
        plsc.subcore_barrier()

        base = c * (E // 2) + s * (nfull * _C)
        nchunk = jnp.where(s == _NTILE - 1, nlast, nfull)

        def chunk(i, _):
            off = base + i * _C
            pltpu.sync_copy(src_h.at[pl.ds(off, _C)], sidx)
            pltpu.sync_copy(dst_h.at[pl.ds(off, _C)], didx)
            pltpu.async_copy(t1_h.at[sidx], srows, sem).wait()
            pltpu.async_copy(adst_h.at[didx], arows, sem).wait()

            @plsc.parallel_loop(0, _C, unroll=4)
            def edge(e):
                av = srows[e, pl.ds(128, _LANE)]
                bv = arows[e, pl.ds(0, _LANE)]
                t = av + bv
                t = jnp.where(t >= 0.0, t, 0.2 * t)
                w = jnp.exp(t)
                for hh in range(HEADS):
                    sp = _splat(w, hh)
                    stage[e, pl.ds(_LANE * hh, _LANE)] = (
                        sp * srows[e, pl.ds(_LANE * hh, _LANE)])
                stage[e, pl.ds(128, _LANE)] = w

            pltpu.sync_copy(stage, acc.at[didx], add=True)
            return 0

        lax.fori_loop(0, nchunk, chunk, 0)
        plsc.subcore_barrier()
        _copy_out_tile(acc, out_h, c, s)

    return k(t1, adst, src, dst)


def _sc_gat2(t2, adst2, src, dst):
    nfull = (E // _NTILE) // _C
    nlast = (E - 15 * nfull * _C) // _C
    mesh = plsc.VectorSubcoreMesh(core_axis_name="c", subcore_axis_name="s")

    @functools.partial(
        pl.kernel, mesh=mesh, compiler_params=_SC_PARAMS,
        out_type=jax.ShapeDtypeStruct((2, N, _D2), jnp.float32),
        scratch_types=[
            pltpu.VMEM_SHARED((N, _D2), jnp.float32),
            pltpu.VMEM((_C,), jnp.int32),
            pltpu.VMEM((_C,), jnp.int32),
            pltpu.VMEM((_C,), jnp.int32),
            pltpu.VMEM((_C, _D2), jnp.float32),
            pltpu.VMEM((_C, _LANE), jnp.float32),
            pltpu.VMEM((_C, _D2), jnp.float32),
            pltpu.SemaphoreType.DMA,
        ],
    )
    def k(t2_h, adst_h, src_h, dst_h, out_h, acc, sidx, didx, gidx, srows,
          arows, stage, sem):
        c = lax.axis_index("c")
        s = lax.axis_index("s")
        _zero_tile_slice(stage, acc, s, _D2)
        plsc.subcore_barrier()

        base = s * (nfull * _C)
        nchunk = jnp.where(s == _NTILE - 1, nlast, nfull)
        voff = c * N
        lane4 = jnp.arange(_LANE, dtype=jnp.int32) == 4

        def chunk(i, _):
            off = base + i * _C
            pltpu.sync_copy(src_h.at[pl.ds(off, _C)], sidx)
            pltpu.sync_copy(dst_h.at[pl.ds(off, _C)], didx)

            @plsc.parallel_loop(0, _C // _LANE, unroll=4)
            def shift(v):
                gidx[pl.ds(_LANE * v, _LANE)] = (
                    sidx[pl.ds(_LANE * v, _LANE)] + voff)

            pltpu.async_copy(t2_h.at[gidx], srows, sem).wait()

            @plsc.parallel_loop(0, _C // _LANE, unroll=4)
            def shiftd(v):
                gidx[pl.ds(_LANE * v, _LANE)] = (
                    didx[pl.ds(_LANE * v, _LANE)] + voff)

            pltpu.async_copy(adst_h.at[gidx], arows, sem).wait()

            @plsc.parallel_loop(0, _C, unroll=4)
            def edge(e):
                rv6 = srows[e, pl.ds(96, _LANE)]
                t = rv6 + arows[e, pl.ds(0, _LANE)]
                t = jnp.where(t >= 0.0, t, 0.2 * t)
                w = jnp.exp(t)
                sp = _splat(w, 4)
                for kk in range(7):
                    sv = sp * srows[e, pl.ds(_LANE * kk, _LANE)]
                    if kk == 6:
                        sv = jnp.where(lane4, sp, sv)
                    stage[e, pl.ds(_LANE * kk, _LANE)] = sv

            pltpu.sync_copy(stage, acc.at[didx], add=True)
            return 0

        lax.fori_loop(0, nchunk, chunk, 0)
        plsc.subcore_barrier()
        _copy_out_tile(acc, out_h, c, s)

    return k(t2, adst2, src, dst)


def kernel(x, edge_index, cluster_id, cluster_index, W1, att_src1, att_dst1, b1,
           W2, att_src2, att_dst2, b2):
    src = edge_index[0].astype(jnp.int32)
    dst = edge_index[1].astype(jnp.int32)

    h = _matmul(x, W1)
    hr = h.reshape(N, HEADS, NHID)
    a_src = jnp.sum(hr * att_src1, axis=-1)
    a_dst = jnp.sum(hr * att_dst1, axis=-1)
    zpad = jnp.zeros((N, 8), jnp.float32)
    t1 = jnp.concatenate([h, a_src, zpad], axis=1)
    adst_tab = jnp.concatenate([a_dst, zpad], axis=1)

    acc = _sc_gat1(t1, adst_tab, src, dst)
    accs = acc[0] + acc[1]
    w_self = jnp.exp(jax.nn.leaky_relu(a_src + a_dst, negative_slope=0.2))
    num = accs[:, :128].reshape(N, HEADS, NHID) + w_self[:, :, None] * hr
    den = accs[:, 128:136] + w_self
    h1 = (num / (den[:, :, None] + 1e-16)).reshape(N, HEADS * NHID) + b1
    h1 = jax.nn.elu(h1)

    sel_id = cluster_id[cluster_index]
    sel_x = h1[cluster_index]
    cf = (sel_id.T @ sel_x) / sel_id.sum(0)[:, None]
    cidx = jnp.argmax(cluster_id, axis=1)

    W2a, W2b = W2[:128], W2[128:]
    hA = _matmul(h1, W2a)
    hB = _matmul(h1, W2b)
    cfA = cf @ W2a
    cfB = cf @ W2b
    h2lo = hA + cfB[cidx]
    h2hi = hB + cfA[cidx]
    vs2 = att_src2.reshape(NCLASS * NCLASS)
    vd2 = att_dst2.reshape(NCLASS * NCLASS)
    as_lo = h2lo @ vs2
    as_hi = h2hi @ vs2
    ad_lo = h2lo @ vd2
    ad_hi = h2hi @ vd2

    zpad2 = jnp.zeros((N, 11), jnp.float32)
    t2 = jnp.concatenate([
        jnp.concatenate([h2lo, as_lo[:, None], zpad2], axis=1),
        jnp.concatenate([h2hi, as_hi[:, None], zpad2], axis=1)], axis=0)
    adst2_tab = jnp.concatenate([
        jnp.broadcast_to(ad_lo[:, None], (N, 16)),
        jnp.broadcast_to(ad_hi[:, None], (N, 16))], axis=0)

    acc2 = _sc_gat2(t2, adst2_tab, src, dst)

    def finish(acch, h2, a_s, a_d):
        ws = jnp.exp(jax.nn.leaky_relu(a_s + a_d, negative_slope=0.2))
        numv = acch[:, :100] + ws[:, None] * h2
        denv = acch[:, 100:101] + ws[:, None]
        return numv / (denv + 1e-16)

    out_lo = finish(acc2[0], h2lo, as_lo, ad_lo)
    out_hi = finish(acc2[1], h2hi, as_hi, ad_hi)
    return jnp.concatenate([out_lo, out_hi], axis=0) + b2


# speedup vs baseline: 61.8183x; 1.3109x over previous
import functools

import jax
import jax.numpy as jnp
from jax import lax
from jax.experimental import pallas as pl
from jax.experimental.pallas import tpu as pltpu
from jax.experimental.pallas import tpu_sc as plsc

N = 10000
E = 320000
NFEAT = 128
NHID = 16
HEADS = 8
NCLASS = 10
CLUSTER = 20
NTRAIN = 5000

_C1 = 64
_C2 = 80
_D1 = 144
_D2 = 112
_NTILE = 16
_RPT = 624
_TAIL = N - _NTILE * _RPT
_LANE = 16
_SC_PARAMS = pltpu.CompilerParams(use_tc_tiling_on_sc=False)


def _mm_kernel(x_ref, w_ref, o_ref):
    o_ref[...] = jnp.dot(x_ref[...], w_ref[...], preferred_element_type=jnp.float32)


def _matmul(x, w):
    m, k = x.shape
    _, n = w.shape
    bm = 1000
    return pl.pallas_call(
        _mm_kernel,
        grid=(m // bm,),
        in_specs=[
            pl.BlockSpec((bm, k), lambda i: (i, 0)),
            pl.BlockSpec((k, n), lambda i: (0, 0)),
        ],
        out_specs=pl.BlockSpec((bm, n), lambda i: (i, 0)),
        out_shape=jax.ShapeDtypeStruct((m, n), jnp.float32),
    )(x, w)


def _splat(v, lane):
    idx = jnp.full((_LANE, 1), lane, dtype=jnp.int32)
    return lax.gather(
        v, idx,
        lax.GatherDimensionNumbers(offset_dims=(), collapsed_slice_dims=(0,),
                                   start_index_map=(0,)),
        slice_sizes=(1,), mode=lax.GatherScatterMode.PROMISE_IN_BOUNDS)


def _zero_tile_slice(stage, acc, s, d, cc):
    zv = jnp.zeros((_LANE,), jnp.float32)

    def zrow(i, _):
        for hh in range(d // _LANE):
            stage[i, pl.ds(_LANE * hh, _LANE)] = zv
        return 0

    lax.fori_loop(0, cc, zrow, 0)
    r0 = s * _RPT
    nfull = _RPT // cc
    rem = _RPT - nfull * cc

    def zcp(j, _):
        pltpu.sync_copy(stage, acc.at[pl.ds(r0 + j * cc, cc)])
        return 0

    lax.fori_loop(0, nfull, zcp, 0)
    if rem:
        pltpu.sync_copy(stage.at[pl.ds(0, rem)], acc.at[pl.ds(r0 + nfull * cc, rem)])

    @pl.when(s == _NTILE - 1)
    def _():
        pltpu.sync_copy(stage.at[pl.ds(0, _TAIL)],
                        acc.at[pl.ds(_NTILE * _RPT, _TAIL)])


def _copy_out_tile(acc, out_h, c, s):
    r0 = s * _RPT
    pltpu.sync_copy(acc.at[pl.ds(r0, _RPT)], out_h.at[c, pl.ds(r0, _RPT)])

    @pl.when(s == _NTILE - 1)
    def _():
        pltpu.sync_copy(acc.at[pl.ds(_NTILE * _RPT, _TAIL)],
                        out_h.at[c, pl.ds(_NTILE * _RPT, _TAIL)])


def _sc_gat1(t1, adst, src, dst):
    nfull = (E // 2 // _NTILE) // _C1
    nlast = (E // 2 - 15 * nfull * _C1) // _C1
    mesh = plsc.VectorSubcoreMesh(core_axis_name="c", subcore_axis_name="s")

    bank = [
        pltpu.VMEM((_C1,), jnp.int32),
        pltpu.VMEM((_C1,), jnp.int32),
        pltpu.VMEM((_C1, _D1), jnp.float32),
        pltpu.VMEM((_C1, _LANE), jnp.float32),
        pltpu.VMEM((_C1, _D1), jnp.float32),
        pltpu.SemaphoreType.DMA,
    ]

    @functools.partial(
        pl.kernel, mesh=mesh, compiler_params=_SC_PARAMS,
        out_type=jax.ShapeDtypeStruct((2, N, _D1), jnp.float32),
        scratch_types=[pltpu.VMEM_SHARED((N, _D1), jnp.float32)] + bank + bank,
    )
    def k(t1_h, adst_h, src_h, dst_h, out_h, acc,
          sidx0, didx0, srows0, arows0, stage0, sem0,
          sidx1, didx1, srows1, arows1, stage1, sem1):
        c = lax.axis_index("c")
        s = lax.axis_index("s")
        banks = [(sidx0, didx0, srows0, arows0, stage0, sem0),
                 (sidx1, didx1, srows1, arows1, stage1, sem1)]

        base = c * (E // 2) + s * (nfull * _C1)
        npair = jnp.where(s == _NTILE - 1, nlast // 2, nfull // 2)

        def load_and_start(b, chunk_idx):
            sidx, didx, srows, arows, _, sem = banks[b]
            off = base + chunk_idx * _C1
            pltpu.sync_copy(src_h.at[pl.ds(off, _C1)], sidx)
            pltpu.sync_copy(dst_h.at[pl.ds(off, _C1)], didx)
            pltpu.async_copy(t1_h.at[sidx], srows, sem)
            pltpu.async_copy(adst_h.at[didx], arows, sem)

        def wait_gathers(b):
            sidx, didx, srows, arows, _, sem = banks[b]
            pltpu.make_async_copy(t1_h.at[sidx], srows, sem).wait()
            pltpu.make_async_copy(adst_h.at[didx], arows, sem).wait()

        def compute_scatter(b):
            _, didx, srows, arows, stage, _ = banks[b]

            @plsc.parallel_loop(0, _C1, unroll=8)
            def edge(e):
                av = srows[e, pl.ds(128, _LANE)]
                bv = arows[e, pl.ds(0, _LANE)]
                t = av + bv
                t = jnp.where(t >= 0.0, t, 0.2 * t)
                w = jnp.exp(t)
                for hh in range(HEADS):
                    sp = _splat(w, hh)
                    stage[e, pl.ds(_LANE * hh, _LANE)] = (
                        sp * srows[e, pl.ds(_LANE * hh, _LANE)])
                stage[e, pl.ds(128, _LANE)] = w

            pltpu.sync_copy(stage, acc.at[didx], add=True)

        load_and_start(0, 0)
        _zero_tile_slice(stage1, acc, s, _D1, _C1)
        plsc.subcore_barrier()

        def pair(i, _):
            load_and_start(1, 2 * i + 1)
            wait_gathers(0)
            compute_scatter(0)

            @pl.when(i + 1 < npair)
            def _():
                load_and_start(0, 2 * i + 2)

            wait_gathers(1)
            compute_scatter(1)
            return 0

        lax.fori_loop(0, npair, pair, 0)
        plsc.subcore_barrier()
        _copy_out_tile(acc, out_h, c, s)

    return k(t1, adst, src, dst)


def _sc_gat2(t2, adst2, src, dst):
    nfull = (E // _NTILE) // _C2
    mesh = plsc.VectorSubcoreMesh(core_axis_name="c", subcore_axis_name="s")

    bank = [
        pltpu.VMEM((_C2,), jnp.int32),
        pltpu.VMEM((_C2,), jnp.int32),
        pltpu.VMEM((_C2,), jnp.int32),
        pltpu.VMEM((_C2, _D2), jnp.float32),
        pltpu.VMEM((_C2, _LANE), jnp.float32),
        pltpu.VMEM((_C2, _D2), jnp.float32),
        pltpu.SemaphoreType.DMA,
    ]

    @functools.partial(
        pl.kernel, mesh=mesh, compiler_params=_SC_PARAMS,
        out_type=jax.ShapeDtypeStruct((2, N, _D2), jnp.float32),
        scratch_types=[pltpu.VMEM_SHARED((N, _D2), jnp.float32)] + bank + bank,
    )
    def k(t2_h, adst_h, src_h, dst_h, out_h, acc,
          gidx0, didx0, gdidx0, srows0, arows0, stage0, sem0,
          gidx1, didx1, gdidx1, srows1, arows1, stage1, sem1):
        c = lax.axis_index("c")
        s = lax.axis_index("s")
        banks = [(gidx0, didx0, gdidx0, srows0, arows0, stage0, sem0),
                 (gidx1, didx1, gdidx1, srows1, arows1, stage1, sem1)]

        base = s * (nfull * _C2)
        npair = nfull // 2
        voff = c * N
        lane4 = jnp.arange(_LANE, dtype=jnp.int32) == 4

        def load_and_start(b, chunk_idx):
            gidx, didx, gdidx, srows, arows, _, sem = banks[b]
            off = base + chunk_idx * _C2
            pltpu.sync_copy(src_h.at[pl.ds(off, _C2)], gidx)
            pltpu.sync_copy(dst_h.at[pl.ds(off, _C2)], didx)

            @plsc.parallel_loop(0, _C2 // _LANE, unroll=4)
            def shift(v):
                sl = pl.ds(_LANE * v, _LANE)
                gidx[sl] = gidx[sl] + voff
                gdidx[sl] = didx[sl] + voff

            pltpu.async_copy(t2_h.at[gidx], srows, sem)
            pltpu.async_copy(adst_h.at[gdidx], arows, sem)

        def wait_gathers(b):
            gidx, _, gdidx, srows, arows, _, sem = banks[b]
            pltpu.make_async_copy(t2_h.at[gidx], srows, sem).wait()
            pltpu.make_async_copy(adst_h.at[gdidx], arows, sem).wait()

        def compute_scatter(b):
            _, didx, _, srows, arows, stage, _ = banks[b]

            @plsc.parallel_loop(0, _C2, unroll=8)
            def edge(e):
                rv6 = srows[e, pl.ds(96, _LANE)]
                t = rv6 + arows[e, pl.ds(0, _LANE)]
                t = jnp.where(t >= 0.0, t, 0.2 * t)
                w = jnp.exp(t)
                sp = _splat(w, 4)
                for kk in range(7):
                    sv = sp * srows[e, pl.ds(_LANE * kk, _LANE)]
                    if kk == 6:
                        sv = jnp.where(lane4, sp, sv)
                    stage[e, pl.ds(_LANE * kk, _LANE)] = sv

            pltpu.sync_copy(stage, acc.at[didx], add=True)

        load_and_start(0, 0)
        _zero_tile_slice(stage1, acc, s, _D2, _C2)
        plsc.subcore_barrier()

        def pair(i, _):
            load_and_start(1, 2 * i + 1)
            wait_gathers(0)
            compute_scatter(0)

            @pl.when(i + 1 < npair)
            def _():
                load_and_start(0, 2 * i + 2)

            wait_gathers(1)
            compute_scatter(1)
            return 0

        lax.fori_loop(0, npair, pair, 0)
        plsc.subcore_barrier()
        _copy_out_tile(acc, out_h, c, s)

    return k(t2, adst2, src, dst)


def kernel(x, edge_index, cluster_id, cluster_index, W1, att_src1, att_dst1, b1,
           W2, att_src2, att_dst2, b2):
    src = edge_index[0].astype(jnp.int32)
    dst = edge_index[1].astype(jnp.int32)

    h = _matmul(x, W1)
    hr = h.reshape(N, HEADS, NHID)
    a_src = jnp.sum(hr * att_src1, axis=-1)
    a_dst = jnp.sum(hr * att_dst1, axis=-1)
    zpad = jnp.zeros((N, 8), jnp.float32)
    t1 = jnp.concatenate([h, a_src, zpad], axis=1)
    adst_tab = jnp.concatenate([a_dst, zpad], axis=1)

    acc = _sc_gat1(t1, adst_tab, src, dst)
    accs = acc[0] + acc[1]
    w_self = jnp.exp(jax.nn.leaky_relu(a_src + a_dst, negative_slope=0.2))
    num = accs[:, :128].reshape(N, HEADS, NHID) + w_self[:, :, None] * hr
    den = accs[:, 128:136] + w_self
    h1 = (num / (den[:, :, None] + 1e-16)).reshape(N, HEADS * NHID) + b1
    h1 = jax.nn.elu(h1)

    sel_id = cluster_id[cluster_index]
    sel_x = h1[cluster_index]
    cf = (sel_id.T @ sel_x) / sel_id.sum(0)[:, None]
    cidx = jnp.argmax(cluster_id, axis=1)

    W2a, W2b = W2[:128], W2[128:]
    hA = _matmul(h1, W2a)
    hB = _matmul(h1, W2b)
    cfA = cf @ W2a
    cfB = cf @ W2b
    h2lo = hA + cfB[cidx]
    h2hi = hB + cfA[cidx]
    vs2 = att_src2.reshape(NCLASS * NCLASS)
    vd2 = att_dst2.reshape(NCLASS * NCLASS)
    as_lo = h2lo @ vs2
    as_hi = h2hi @ vs2
    ad_lo = h2lo @ vd2
    ad_hi = h2hi @ vd2

    zpad2 = jnp.zeros((N, 11), jnp.float32)
    t2 = jnp.concatenate([
        jnp.concatenate([h2lo, as_lo[:, None], zpad2], axis=1),
        jnp.concatenate([h2hi, as_hi[:, None], zpad2], axis=1)], axis=0)
    adst2_tab = jnp.concatenate([
        jnp.broadcast_to(ad_lo[:, None], (N, 16)),
        jnp.broadcast_to(ad_hi[:, None], (N, 16))], axis=0)

    acc2 = _sc_gat2(t2, adst2_tab, src, dst)

    def finish(acch, h2, a_s, a_d):
        ws = jnp.exp(jax.nn.leaky_relu(a_s + a_d, negative_slope=0.2))
        numv = acch[:, :100] + ws[:, None] * h2
        denv = acch[:, 100:101] + ws[:, None]
        return numv / (denv + 1e-16)

    out_lo = finish(acc2[0], h2lo, as_lo, ad_lo)
    out_hi = finish(acc2[1], h2hi, as_hi, ad_hi)
    return jnp.concatenate([out_lo, out_hi], axis=0) + b2


# speedup vs baseline: 64.8867x; 1.0496x over previous
import functools

import jax
import jax.numpy as jnp
from jax import lax
from jax.experimental import pallas as pl
from jax.experimental.pallas import tpu as pltpu
from jax.experimental.pallas import tpu_sc as plsc

N = 10000
E = 320000
NFEAT = 128
NHID = 16
HEADS = 8
NCLASS = 10
CLUSTER = 20
NTRAIN = 5000

_C1 = 64
_C2 = 80
_D1 = 144
_D2 = 112
_NTILE = 16
_RPT = 624
_TAIL = N - _NTILE * _RPT
_LANE = 16
_SC_PARAMS = pltpu.CompilerParams(use_tc_tiling_on_sc=False)


_BM = 1000


def _lrelu(t):
    return jnp.where(t >= 0.0, t, 0.2 * t)


def _tc_pre1(x, W1, att_s, att_d):
    def body(x_ref, w_ref, as_ref, ad_ref, t1_ref, adst_ref):
        h = jnp.dot(x_ref[...], w_ref[...], preferred_element_type=jnp.float32)
        hr = h.reshape(_BM, HEADS, NHID)
        asb = jnp.sum(hr * as_ref[...], axis=-1)
        adb = jnp.sum(hr * ad_ref[...], axis=-1)
        z8 = jnp.zeros((_BM, 8), jnp.float32)
        t1_ref[...] = jnp.concatenate([h, asb, z8], axis=1)
        adst_ref[...] = jnp.concatenate([adb, z8], axis=1)

    return pl.pallas_call(
        body,
        grid=(N // _BM,),
        in_specs=[
            pl.BlockSpec((_BM, NFEAT), lambda i: (i, 0)),
            pl.BlockSpec((NFEAT, NFEAT), lambda i: (0, 0)),
            pl.BlockSpec((1, HEADS, NHID), lambda i: (0, 0, 0)),
            pl.BlockSpec((1, HEADS, NHID), lambda i: (0, 0, 0)),
        ],
        out_specs=[
            pl.BlockSpec((_BM, _D1), lambda i: (i, 0)),
            pl.BlockSpec((_BM, _LANE), lambda i: (i, 0)),
        ],
        out_shape=[
            jax.ShapeDtypeStruct((N, _D1), jnp.float32),
            jax.ShapeDtypeStruct((N, _LANE), jnp.float32),
        ],
    )(x, W1, att_s, att_d)


def _tc_post1(acc, t1, adst, b1):
    def body(acc_ref, t1_ref, adst_ref, b1_ref, o_ref):
        accs = acc_ref[0] + acc_ref[1]
        h = t1_ref[:, :128]
        a_s = t1_ref[:, 128:136]
        a_d = adst_ref[:, :8]
        ws = jnp.exp(_lrelu(a_s + a_d))
        hr = h.reshape(_BM, HEADS, NHID)
        num = accs[:, :128].reshape(_BM, HEADS, NHID) + ws[:, :, None] * hr
        den = accs[:, 128:136] + ws
        h1 = (num / (den[:, :, None] + 1e-16)).reshape(_BM, 128) + b1_ref[...]
        o_ref[...] = jnp.where(h1 > 0.0, h1, jnp.exp(h1) - 1.0)

    return pl.pallas_call(
        body,
        grid=(N // _BM,),
        in_specs=[
            pl.BlockSpec((2, _BM, _D1), lambda i: (0, i, 0)),
            pl.BlockSpec((_BM, _D1), lambda i: (i, 0)),
            pl.BlockSpec((_BM, _LANE), lambda i: (i, 0)),
            pl.BlockSpec((1, 128), lambda i: (0, 0)),
        ],
        out_specs=pl.BlockSpec((_BM, 128), lambda i: (i, 0)),
        out_shape=jax.ShapeDtypeStruct((N, 128), jnp.float32),
    )(acc, t1, adst, b1.reshape(1, 128))


def _tc_pre2(h1, W2a, W2b, x1b, x1a, vs2, vd2):
    def body(h1_ref, wa_ref, wb_ref, x1b_ref, x1a_ref, vs_ref, vd_ref,
             t2_ref, adst_ref):
        lo = jnp.dot(h1_ref[...], wa_ref[...],
                     preferred_element_type=jnp.float32) + x1b_ref[...]
        hi = jnp.dot(h1_ref[...], wb_ref[...],
                     preferred_element_type=jnp.float32) + x1a_ref[...]
        z11 = jnp.zeros((_BM, 11), jnp.float32)
        for j, h2 in enumerate((lo, hi)):
            a_s = jnp.sum(h2 * vs_ref[...], axis=1, keepdims=True)
            a_d = jnp.sum(h2 * vd_ref[...], axis=1, keepdims=True)
            t2_ref[j] = jnp.concatenate([h2, a_s, z11], axis=1)
            adst_ref[j] = jnp.broadcast_to(a_d, (_BM, _LANE))

    return pl.pallas_call(
        body,
        grid=(N // _BM,),
        in_specs=[
            pl.BlockSpec((_BM, 128), lambda i: (i, 0)),
            pl.BlockSpec((128, 100), lambda i: (0, 0)),
            pl.BlockSpec((128, 100), lambda i: (0, 0)),
            pl.BlockSpec((_BM, 100), lambda i: (i, 0)),
            pl.BlockSpec((_BM, 100), lambda i: (i, 0)),
            pl.BlockSpec((1, 100), lambda i: (0, 0)),
            pl.BlockSpec((1, 100), lambda i: (0, 0)),
        ],
        out_specs=[
            pl.BlockSpec((2, _BM, _D2), lambda i: (0, i, 0)),
            pl.BlockSpec((2, _BM, _LANE), lambda i: (0, i, 0)),
        ],
        out_shape=[
            jax.ShapeDtypeStruct((2, N, _D2), jnp.float32),
            jax.ShapeDtypeStruct((2, N, _LANE), jnp.float32),
        ],
    )(h1, W2a, W2b, x1b, x1a, vs2.reshape(1, 100), vd2.reshape(1, 100))


def _tc_post2(acc2, t2, adst2, b2):
    def body(acc_ref, t2_ref, adst_ref, b2_ref, o_ref):
        for j in range(2):
            h2 = t2_ref[j, :, :100]
            a_s = t2_ref[j, :, 100:101]
            a_d = adst_ref[j, :, 0:1]
            ws = jnp.exp(_lrelu(a_s + a_d))
            num = acc_ref[j, :, :100] + ws * h2
            den = acc_ref[j, :, 100:101] + ws
            o_ref[j] = num / (den + 1e-16) + b2_ref[...]

    return pl.pallas_call(
        body,
        grid=(N // _BM,),
        in_specs=[
            pl.BlockSpec((2, _BM, _D2), lambda i: (0, i, 0)),
            pl.BlockSpec((2, _BM, _D2), lambda i: (0, i, 0)),
            pl.BlockSpec((2, _BM, _LANE), lambda i: (0, i, 0)),
            pl.BlockSpec((1, 100), lambda i: (0, 0)),
        ],
        out_specs=pl.BlockSpec((2, _BM, 100), lambda i: (0, i, 0)),
        out_shape=jax.ShapeDtypeStruct((2, N, 100), jnp.float32),
    )(acc2, t2, adst2, b2.reshape(1, 100))


def _splat(v, lane):
    idx = jnp.full((_LANE, 1), lane, dtype=jnp.int32)
    return lax.gather(
        v, idx,
        lax.GatherDimensionNumbers(offset_dims=(), collapsed_slice_dims=(0,),
                                   start_index_map=(0,)),
        slice_sizes=(1,), mode=lax.GatherScatterMode.PROMISE_IN_BOUNDS)


def _zero_tile_slice(stage, acc, s, d, cc):
    zv = jnp.zeros((_LANE,), jnp.float32)

    def zrow(i, _):
        for hh in range(d // _LANE):
            stage[i, pl.ds(_LANE * hh, _LANE)] = zv
        return 0

    lax.fori_loop(0, cc, zrow, 0)
    r0 = s * _RPT
    nfull = _RPT // cc
    rem = _RPT - nfull * cc

    def zcp(j, _):
        pltpu.sync_copy(stage, acc.at[pl.ds(r0 + j * cc, cc)])
        return 0

    lax.fori_loop(0, nfull, zcp, 0)
    if rem:
        pltpu.sync_copy(stage.at[pl.ds(0, rem)], acc.at[pl.ds(r0 + nfull * cc, rem)])

    @pl.when(s == _NTILE - 1)
    def _():
        pltpu.sync_copy(stage.at[pl.ds(0, _TAIL)],
                        acc.at[pl.ds(_NTILE * _RPT, _TAIL)])


def _copy_out_tile(acc, out_h, c, s):
    r0 = s * _RPT
    pltpu.sync_copy(acc.at[pl.ds(r0, _RPT)], out_h.at[c, pl.ds(r0, _RPT)])

    @pl.when(s == _NTILE - 1)
    def _():
        pltpu.sync_copy(acc.at[pl.ds(_NTILE * _RPT, _TAIL)],
                        out_h.at[c, pl.ds(_NTILE * _RPT, _TAIL)])


def _sc_gat1(t1, adst, src, dst):
    nfull = (E // 2 // _NTILE) // _C1
    nlast = (E // 2 - 15 * nfull * _C1) // _C1
    mesh = plsc.VectorSubcoreMesh(core_axis_name="c", subcore_axis_name="s")

    bank = [
        pltpu.VMEM((_C1,), jnp.int32),
        pltpu.VMEM((_C1,), jnp.int32),
        pltpu.VMEM((_C1, _D1), jnp.float32),
        pltpu.VMEM((_C1, _LANE), jnp.float32),
        pltpu.VMEM((_C1, _D1), jnp.float32),
        pltpu.SemaphoreType.DMA,
    ]

    @functools.partial(
        pl.kernel, mesh=mesh, compiler_params=_SC_PARAMS,
        out_type=jax.ShapeDtypeStruct((2, N, _D1), jnp.float32),
        scratch_types=[pltpu.VMEM_SHARED((N, _D1), jnp.float32)] + bank + bank,
    )
    def k(t1_h, adst_h, src_h, dst_h, out_h, acc,
          sidx0, didx0, srows0, arows0, stage0, sem0,
          sidx1, didx1, srows1, arows1, stage1, sem1):
        c = lax.axis_index("c")
        s = lax.axis_index("s")
        banks = [(sidx0, didx0, srows0, arows0, stage0, sem0),
                 (sidx1, didx1, srows1, arows1, stage1, sem1)]

        base = c * (E // 2) + s * (nfull * _C1)
        npair = jnp.where(s == _NTILE - 1, nlast // 2, nfull // 2)

        def load_and_start(b, chunk_idx):
            sidx, didx, srows, arows, _, sem = banks[b]
            off = base + chunk_idx * _C1
            pltpu.sync_copy(src_h.at[pl.ds(off, _C1)], sidx)
            pltpu.sync_copy(dst_h.at[pl.ds(off, _C1)], didx)
            pltpu.async_copy(t1_h.at[sidx], srows, sem)
            pltpu.async_copy(adst_h.at[didx], arows, sem)

        def wait_gathers(b):
            sidx, didx, srows, arows, _, sem = banks[b]
            pltpu.make_async_copy(t1_h.at[sidx], srows, sem).wait()
            pltpu.make_async_copy(adst_h.at[didx], arows, sem).wait()

        def compute_scatter(b):
            _, didx, srows, arows, stage, _ = banks[b]

            @plsc.parallel_loop(0, _C1, unroll=8)
            def edge(e):
                av = srows[e, pl.ds(128, _LANE)]
                bv = arows[e, pl.ds(0, _LANE)]
                t = av + bv
                t = jnp.where(t >= 0.0, t, 0.2 * t)
                w = jnp.exp(t)
                for hh in range(HEADS):
                    sp = _splat(w, hh)
                    stage[e, pl.ds(_LANE * hh, _LANE)] = (
                        sp * srows[e, pl.ds(_LANE * hh, _LANE)])
                stage[e, pl.ds(128, _LANE)] = w

            pltpu.sync_copy(stage, acc.at[didx], add=True)

        load_and_start(0, 0)
        _zero_tile_slice(stage1, acc, s, _D1, _C1)
        plsc.subcore_barrier()

        def pair(i, _):
            load_and_start(1, 2 * i + 1)
            wait_gathers(0)
            compute_scatter(0)

            @pl.when(i + 1 < npair)
            def _():
                load_and_start(0, 2 * i + 2)

            wait_gathers(1)
            compute_scatter(1)
            return 0

        lax.fori_loop(0, npair, pair, 0)
        plsc.subcore_barrier()
        _copy_out_tile(acc, out_h, c, s)

    return k(t1, adst, src, dst)


def _sc_gat2(t2, adst2, src, dst):
    nfull = (E // _NTILE) // _C2
    mesh = plsc.VectorSubcoreMesh(core_axis_name="c", subcore_axis_name="s")

    bank = [
        pltpu.VMEM((_C2,), jnp.int32),
        pltpu.VMEM((_C2,), jnp.int32),
        pltpu.VMEM((_C2,), jnp.int32),
        pltpu.VMEM((_C2, _D2), jnp.float32),
        pltpu.VMEM((_C2, _LANE), jnp.float32),
        pltpu.VMEM((_C2, _D2), jnp.float32),
        pltpu.SemaphoreType.DMA,
    ]

    @functools.partial(
        pl.kernel, mesh=mesh, compiler_params=_SC_PARAMS,
        out_type=jax.ShapeDtypeStruct((2, N, _D2), jnp.float32),
        scratch_types=[pltpu.VMEM_SHARED((N, _D2), jnp.float32)] + bank + bank,
    )
    def k(t2_h, adst_h, src_h, dst_h, out_h, acc,
          gidx0, didx0, gdidx0, srows0, arows0, stage0, sem0,
          gidx1, didx1, gdidx1, srows1, arows1, stage1, sem1):
        c = lax.axis_index("c")
        s = lax.axis_index("s")
        banks = [(gidx0, didx0, gdidx0, srows0, arows0, stage0, sem0),
                 (gidx1, didx1, gdidx1, srows1, arows1, stage1, sem1)]

        base = s * (nfull * _C2)
        npair = nfull // 2
        voff = c * N
        lane4 = jnp.arange(_LANE, dtype=jnp.int32) == 4

        def load_and_start(b, chunk_idx):
            gidx, didx, gdidx, srows, arows, _, sem = banks[b]
            off = base + chunk_idx * _C2
            pltpu.sync_copy(src_h.at[pl.ds(off, _C2)], gidx)
            pltpu.sync_copy(dst_h.at[pl.ds(off, _C2)], didx)

            @plsc.parallel_loop(0, _C2 // _LANE, unroll=4)
            def shift(v):
                sl = pl.ds(_LANE * v, _LANE)
                gidx[sl] = gidx[sl] + voff
                gdidx[sl] = didx[sl] + voff

            pltpu.async_copy(t2_h.at[gidx], srows, sem)
            pltpu.async_copy(adst_h.at[gdidx], arows, sem)

        def wait_gathers(b):
            gidx, _, gdidx, srows, arows, _, sem = banks[b]
            pltpu.make_async_copy(t2_h.at[gidx], srows, sem).wait()
            pltpu.make_async_copy(adst_h.at[gdidx], arows, sem).wait()

        def compute_scatter(b):
            _, didx, _, srows, arows, stage, _ = banks[b]

            @plsc.parallel_loop(0, _C2, unroll=8)
            def edge(e):
                rv6 = srows[e, pl.ds(96, _LANE)]
                t = rv6 + arows[e, pl.ds(0, _LANE)]
                t = jnp.where(t >= 0.0, t, 0.2 * t)
                w = jnp.exp(t)
                sp = _splat(w, 4)
                for kk in range(7):
                    sv = sp * srows[e, pl.ds(_LANE * kk, _LANE)]
                    if kk == 6:
                        sv = jnp.where(lane4, sp, sv)
                    stage[e, pl.ds(_LANE * kk, _LANE)] = sv

            pltpu.sync_copy(stage, acc.at[didx], add=True)

        load_and_start(0, 0)
        _zero_tile_slice(stage1, acc, s, _D2, _C2)
        plsc.subcore_barrier()

        def pair(i, _):
            load_and_start(1, 2 * i + 1)
            wait_gathers(0)
            compute_scatter(0)

            @pl.when(i + 1 < npair)
            def _():
                load_and_start(0, 2 * i + 2)

            wait_gathers(1)
            compute_scatter(1)
            return 0

        lax.fori_loop(0, npair, pair, 0)
        plsc.subcore_barrier()
        _copy_out_tile(acc, out_h, c, s)

    return k(t2, adst2, src, dst)


def kernel(x, edge_index, cluster_id, cluster_index, W1, att_src1, att_dst1, b1,
           W2, att_src2, att_dst2, b2):
    src = edge_index[0].astype(jnp.int32)
    dst = edge_index[1].astype(jnp.int32)

    t1, adst_tab = _tc_pre1(x, W1, att_src1, att_dst1)
    acc = _sc_gat1(t1, adst_tab, src, dst)
    h1 = _tc_post1(acc, t1, adst_tab, b1)

    sel_id = cluster_id[cluster_index]
    sel_x = h1[cluster_index]
    cf = (sel_id.T @ sel_x) / sel_id.sum(0)[:, None]
    cidx = jnp.argmax(cluster_id, axis=1)

    W2a, W2b = W2[:128], W2[128:]
    cfA = cf @ W2a
    cfB = cf @ W2b
    x1b = cfB[cidx]
    x1a = cfA[cidx]
    t2, adst2_tab = _tc_pre2(h1, W2a, W2b, x1b, x1a,
                             att_src2.reshape(-1), att_dst2.reshape(-1))

    acc2 = _sc_gat2(t2.reshape(2 * N, _D2),
                    adst2_tab.reshape(2 * N, _LANE), src, dst)

    out = _tc_post2(acc2, t2, adst2_tab, b2)
    return out.reshape(2 * N, NCLASS * NCLASS)


# speedup vs baseline: 75.2307x; 1.1594x over previous
import functools

import jax
import jax.numpy as jnp
from jax import lax
from jax.experimental import pallas as pl
from jax.experimental.pallas import tpu as pltpu
from jax.experimental.pallas import tpu_sc as plsc

N = 10000
E = 320000
NFEAT = 128
NHID = 16
HEADS = 8
NCLASS = 10
CLUSTER = 20
NTRAIN = 5000

_C1 = 64
_C2 = 80
_D1 = 144
_D2 = 112
_NTILE = 16
_RPT = 624
_TAIL = N - _NTILE * _RPT
_LANE = 16
_SC_PARAMS = pltpu.CompilerParams(use_tc_tiling_on_sc=False)


_BM = 1000


def _lrelu(t):
    return jnp.where(t >= 0.0, t, 0.2 * t)


def _tc_pre1(x, W1, att_s, att_d):
    def body(x_ref, w_ref, as_ref, ad_ref, t1_ref, adst_ref):
        h = jnp.dot(x_ref[...], w_ref[...], preferred_element_type=jnp.float32)
        hr = h.reshape(_BM, HEADS, NHID)
        asb = jnp.sum(hr * as_ref[...], axis=-1)
        adb = jnp.sum(hr * ad_ref[...], axis=-1)
        z8 = jnp.zeros((_BM, 8), jnp.float32)
        t1_ref[...] = jnp.concatenate([h, asb, z8], axis=1)
        adst_ref[...] = jnp.concatenate([adb, z8], axis=1)

    return pl.pallas_call(
        body,
        grid=(N // _BM,),
        in_specs=[
            pl.BlockSpec((_BM, NFEAT), lambda i: (i, 0)),
            pl.BlockSpec((NFEAT, NFEAT), lambda i: (0, 0)),
            pl.BlockSpec((1, HEADS, NHID), lambda i: (0, 0, 0)),
            pl.BlockSpec((1, HEADS, NHID), lambda i: (0, 0, 0)),
        ],
        out_specs=[
            pl.BlockSpec((_BM, _D1), lambda i: (i, 0)),
            pl.BlockSpec((_BM, _LANE), lambda i: (i, 0)),
        ],
        out_shape=[
            jax.ShapeDtypeStruct((N, _D1), jnp.float32),
            jax.ShapeDtypeStruct((N, _LANE), jnp.float32),
        ],
    )(x, W1, att_s, att_d)


def _tc_post1(acc, t1, adst, b1):
    def body(acc_ref, t1_ref, adst_ref, b1_ref, o_ref):
        accs = acc_ref[0] + acc_ref[1]
        h = t1_ref[:, :128]
        a_s = t1_ref[:, 128:136]
        a_d = adst_ref[:, :8]
        ws = jnp.exp(_lrelu(a_s + a_d))
        hr = h.reshape(_BM, HEADS, NHID)
        num = accs[:, :128].reshape(_BM, HEADS, NHID) + ws[:, :, None] * hr
        den = accs[:, 128:136] + ws
        h1 = (num / (den[:, :, None] + 1e-16)).reshape(_BM, 128) + b1_ref[...]
        o_ref[...] = jnp.where(h1 > 0.0, h1, jnp.exp(h1) - 1.0)

    return pl.pallas_call(
        body,
        grid=(N // _BM,),
        in_specs=[
            pl.BlockSpec((2, _BM, _D1), lambda i: (0, i, 0)),
            pl.BlockSpec((_BM, _D1), lambda i: (i, 0)),
            pl.BlockSpec((_BM, _LANE), lambda i: (i, 0)),
            pl.BlockSpec((1, 128), lambda i: (0, 0)),
        ],
        out_specs=pl.BlockSpec((_BM, 128), lambda i: (i, 0)),
        out_shape=jax.ShapeDtypeStruct((N, 128), jnp.float32),
    )(acc, t1, adst, b1.reshape(1, 128))


def _tc_pre2(h1, W2a, W2b, x1b, x1a, vs2, vd2):
    def body(h1_ref, wa_ref, wb_ref, x1b_ref, x1a_ref, vs_ref, vd_ref,
             t2_ref, adst_ref):
        lo = jnp.dot(h1_ref[...], wa_ref[...],
                     preferred_element_type=jnp.float32) + x1b_ref[...]
        hi = jnp.dot(h1_ref[...], wb_ref[...],
                     preferred_element_type=jnp.float32) + x1a_ref[...]
        z11 = jnp.zeros((_BM, 11), jnp.float32)
        for j, h2 in enumerate((lo, hi)):
            a_s = jnp.sum(h2 * vs_ref[...], axis=1, keepdims=True)
            a_d = jnp.sum(h2 * vd_ref[...], axis=1, keepdims=True)
            t2_ref[j] = jnp.concatenate([h2, a_s, z11], axis=1)
            adst_ref[j] = jnp.broadcast_to(a_d, (_BM, _LANE))

    return pl.pallas_call(
        body,
        grid=(N // _BM,),
        in_specs=[
            pl.BlockSpec((_BM, 128), lambda i: (i, 0)),
            pl.BlockSpec((128, 100), lambda i: (0, 0)),
            pl.BlockSpec((128, 100), lambda i: (0, 0)),
            pl.BlockSpec((_BM, 100), lambda i: (i, 0)),
            pl.BlockSpec((_BM, 100), lambda i: (i, 0)),
            pl.BlockSpec((1, 100), lambda i: (0, 0)),
            pl.BlockSpec((1, 100), lambda i: (0, 0)),
        ],
        out_specs=[
            pl.BlockSpec((2, _BM, _D2), lambda i: (0, i, 0)),
            pl.BlockSpec((2, _BM, _LANE), lambda i: (0, i, 0)),
        ],
        out_shape=[
            jax.ShapeDtypeStruct((2, N, _D2), jnp.float32),
            jax.ShapeDtypeStruct((2, N, _LANE), jnp.float32),
        ],
    )(h1, W2a, W2b, x1b, x1a, vs2.reshape(1, 100), vd2.reshape(1, 100))


def _tc_post2(acc2, t2, adst2, b2):
    def body(acc_ref, t2_ref, adst_ref, b2_ref, o_ref):
        for j in range(2):
            h2 = t2_ref[j, :, :100]
            a_s = t2_ref[j, :, 100:101]
            a_d = adst_ref[j, :, 0:1]
            ws = jnp.exp(_lrelu(a_s + a_d))
            num = acc_ref[j, :, :100] + ws * h2
            den = acc_ref[j, :, 100:101] + ws
            o_ref[j] = num / (den + 1e-16) + b2_ref[...]

    return pl.pallas_call(
        body,
        grid=(N // _BM,),
        in_specs=[
            pl.BlockSpec((2, _BM, _D2), lambda i: (0, i, 0)),
            pl.BlockSpec((2, _BM, _D2), lambda i: (0, i, 0)),
            pl.BlockSpec((2, _BM, _LANE), lambda i: (0, i, 0)),
            pl.BlockSpec((1, 100), lambda i: (0, 0)),
        ],
        out_specs=pl.BlockSpec((2, _BM, 100), lambda i: (0, i, 0)),
        out_shape=jax.ShapeDtypeStruct((2, N, 100), jnp.float32),
    )(acc2, t2, adst2, b2.reshape(1, 100))


def _splat(v, lane):
    idx = jnp.full((_LANE, 1), lane, dtype=jnp.int32)
    return lax.gather(
        v, idx,
        lax.GatherDimensionNumbers(offset_dims=(), collapsed_slice_dims=(0,),
                                   start_index_map=(0,)),
        slice_sizes=(1,), mode=lax.GatherScatterMode.PROMISE_IN_BOUNDS)


def _zero_tile_slice(stage, acc, s, d, cc):
    zv = jnp.zeros((_LANE,), jnp.float32)

    def zrow(i, _):
        for hh in range(d // _LANE):
            stage[i, pl.ds(_LANE * hh, _LANE)] = zv
        return 0

    lax.fori_loop(0, cc, zrow, 0)
    r0 = s * _RPT
    nfull = _RPT // cc
    rem = _RPT - nfull * cc

    def zcp(j, _):
        pltpu.sync_copy(stage, acc.at[pl.ds(r0 + j * cc, cc)])
        return 0

    lax.fori_loop(0, nfull, zcp, 0)
    if rem:
        pltpu.sync_copy(stage.at[pl.ds(0, rem)], acc.at[pl.ds(r0 + nfull * cc, rem)])

    @pl.when(s == _NTILE - 1)
    def _():
        pltpu.sync_copy(stage.at[pl.ds(0, _TAIL)],
                        acc.at[pl.ds(_NTILE * _RPT, _TAIL)])


def _copy_out_tile(acc, out_h, c, s):
    r0 = s * _RPT
    pltpu.sync_copy(acc.at[pl.ds(r0, _RPT)], out_h.at[c, pl.ds(r0, _RPT)])

    @pl.when(s == _NTILE - 1)
    def _():
        pltpu.sync_copy(acc.at[pl.ds(_NTILE * _RPT, _TAIL)],
                        out_h.at[c, pl.ds(_NTILE * _RPT, _TAIL)])


def _sc_gat1(t1, adst, src, dst):
    nfull = (E // 2 // _NTILE) // _C1
    nlast = (E // 2 - 15 * nfull * _C1) // _C1
    mesh = plsc.VectorSubcoreMesh(core_axis_name="c", subcore_axis_name="s")

    bank = [
        pltpu.VMEM((_C1,), jnp.int32),
        pltpu.VMEM((_C1,), jnp.int32),
        pltpu.VMEM((_C1,), jnp.int32),
        pltpu.VMEM((_C1, _D1), jnp.float32),
        pltpu.VMEM((_C1, _LANE), jnp.float32),
        pltpu.VMEM((_C1, _D1), jnp.float32),
        pltpu.SemaphoreType.DMA,
        pltpu.SemaphoreType.DMA,
    ]

    @functools.partial(
        pl.kernel, mesh=mesh, compiler_params=_SC_PARAMS,
        out_type=jax.ShapeDtypeStruct((2, N, _D1), jnp.float32),
        scratch_types=[pltpu.VMEM_SHARED((N, _D1), jnp.float32)] + bank + bank,
    )
    def k(t1_h, adst_h, src_h, dst_h, out_h, acc,
          sidx0, didx0, sdidx0, srows0, arows0, stage0, sem0, ssem0,
          sidx1, didx1, sdidx1, srows1, arows1, stage1, sem1, ssem1):
        c = lax.axis_index("c")
        s = lax.axis_index("s")
        banks = [(sidx0, didx0, sdidx0, srows0, arows0, stage0, sem0, ssem0),
                 (sidx1, didx1, sdidx1, srows1, arows1, stage1, sem1, ssem1)]

        base = c * (E // 2) + s * (nfull * _C1)
        npair = jnp.where(s == _NTILE - 1, nlast // 2, nfull // 2)

        def load_and_start(b, chunk_idx):
            sidx, didx, _, srows, arows, _, sem, _ = banks[b]
            off = base + chunk_idx * _C1
            pltpu.sync_copy(src_h.at[pl.ds(off, _C1)], sidx)
            pltpu.sync_copy(dst_h.at[pl.ds(off, _C1)], didx)
            pltpu.async_copy(t1_h.at[sidx], srows, sem)
            pltpu.async_copy(adst_h.at[didx], arows, sem)

        def wait_gathers(b):
            sidx, didx, _, srows, arows, _, sem, _ = banks[b]
            pltpu.make_async_copy(t1_h.at[sidx], srows, sem).wait()
            pltpu.make_async_copy(adst_h.at[didx], arows, sem).wait()

        def wait_scatter(b):
            _, _, sdidx, _, _, stage, _, ssem = banks[b]
            pltpu.make_async_copy(stage, acc.at[sdidx], ssem).wait()

        def compute_scatter(b):
            _, didx, sdidx, srows, arows, stage, _, ssem = banks[b]

            @plsc.parallel_loop(0, _C1, unroll=8)
            def edge(e):
                av = srows[e, pl.ds(128, _LANE)]
                bv = arows[e, pl.ds(0, _LANE)]
                t = av + bv
                t = jnp.where(t >= 0.0, t, 0.2 * t)
                w = jnp.exp(t)
                for hh in range(HEADS):
                    sp = _splat(w, hh)
                    stage[e, pl.ds(_LANE * hh, _LANE)] = (
                        sp * srows[e, pl.ds(_LANE * hh, _LANE)])
                stage[e, pl.ds(128, _LANE)] = w

            @plsc.parallel_loop(0, _C1 // _LANE, unroll=4)
            def cpidx(v):
                sl = pl.ds(_LANE * v, _LANE)
                sdidx[sl] = didx[sl]

            pltpu.async_copy(stage, acc.at[sdidx], sem=ssem, add=True)

        load_and_start(0, 0)
        _zero_tile_slice(stage1, acc, s, _D1, _C1)
        plsc.subcore_barrier()

        def pair(i, _):
            load_and_start(1, 2 * i + 1)
            wait_gathers(0)

            @pl.when(i > 0)
            def _():
                wait_scatter(0)

            compute_scatter(0)

            @pl.when(i + 1 < npair)
            def _():
                load_and_start(0, 2 * i + 2)

            wait_gathers(1)

            @pl.when(i > 0)
            def _():
                wait_scatter(1)

            compute_scatter(1)
            return 0

        lax.fori_loop(0, npair, pair, 0)
        wait_scatter(0)
        wait_scatter(1)
        plsc.subcore_barrier()
        _copy_out_tile(acc, out_h, c, s)

    return k(t1, adst, src, dst)


def _sc_gat2(t2, adst2, src, dst):
    nfull = (E // _NTILE) // _C2
    mesh = plsc.VectorSubcoreMesh(core_axis_name="c", subcore_axis_name="s")

    bank = [
        pltpu.VMEM((_C2,), jnp.int32),
        pltpu.VMEM((_C2,), jnp.int32),
        pltpu.VMEM((_C2,), jnp.int32),
        pltpu.VMEM((_C2,), jnp.int32),
        pltpu.VMEM((_C2, _D2), jnp.float32),
        pltpu.VMEM((_C2, _LANE), jnp.float32),
        pltpu.VMEM((_C2, _D2), jnp.float32),
        pltpu.SemaphoreType.DMA,
        pltpu.SemaphoreType.DMA,
    ]

    @functools.partial(
        pl.kernel, mesh=mesh, compiler_params=_SC_PARAMS,
        out_type=jax.ShapeDtypeStruct((2, N, _D2), jnp.float32),
        scratch_types=[pltpu.VMEM_SHARED((N, _D2), jnp.float32)] + bank + bank,
    )
    def k(t2_h, adst_h, src_h, dst_h, out_h, acc,
          gidx0, didx0, gdidx0, sdidx0, srows0, arows0, stage0, sem0, ssem0,
          gidx1, didx1, gdidx1, sdidx1, srows1, arows1, stage1, sem1, ssem1):
        c = lax.axis_index("c")
        s = lax.axis_index("s")
        banks = [
            (gidx0, didx0, gdidx0, sdidx0, srows0, arows0, stage0, sem0, ssem0),
            (gidx1, didx1, gdidx1, sdidx1, srows1, arows1, stage1, sem1, ssem1)]

        base = s * (nfull * _C2)
        npair = nfull // 2
        voff = c * N
        lane4 = jnp.arange(_LANE, dtype=jnp.int32) == 4

        def load_and_start(b, chunk_idx):
            gidx, didx, gdidx, _, srows, arows, _, sem, _ = banks[b]
            off = base + chunk_idx * _C2
            pltpu.sync_copy(src_h.at[pl.ds(off, _C2)], gidx)
            pltpu.sync_copy(dst_h.at[pl.ds(off, _C2)], didx)

            @plsc.parallel_loop(0, _C2 // _LANE, unroll=4)
            def shift(v):
                sl = pl.ds(_LANE * v, _LANE)
                gidx[sl] = gidx[sl] + voff
                gdidx[sl] = didx[sl] + voff

            pltpu.async_copy(t2_h.at[gidx], srows, sem)
            pltpu.async_copy(adst_h.at[gdidx], arows, sem)

        def wait_gathers(b):
            gidx, _, gdidx, _, srows, arows, _, sem, _ = banks[b]
            pltpu.make_async_copy(t2_h.at[gidx], srows, sem).wait()
            pltpu.make_async_copy(adst_h.at[gdidx], arows, sem).wait()

        def wait_scatter(b):
            _, _, _, sdidx, _, _, stage, _, ssem = banks[b]
            pltpu.make_async_copy(stage, acc.at[sdidx], ssem).wait()

        def compute_scatter(b):
            _, didx, _, sdidx, srows, arows, stage, _, ssem = banks[b]

            @plsc.parallel_loop(0, _C2, unroll=8)
            def edge(e):
                rv6 = srows[e, pl.ds(96, _LANE)]
                t = rv6 + arows[e, pl.ds(0, _LANE)]
                t = jnp.where(t >= 0.0, t, 0.2 * t)
                w = jnp.exp(t)
                sp = _splat(w, 4)
                for kk in range(7):
                    sv = sp * srows[e, pl.ds(_LANE * kk, _LANE)]
                    if kk == 6:
                        sv = jnp.where(lane4, sp, sv)
                    stage[e, pl.ds(_LANE * kk, _LANE)] = sv

            @plsc.parallel_loop(0, _C2 // _LANE, unroll=5)
            def cpidx(v):
                sl = pl.ds(_LANE * v, _LANE)
                sdidx[sl] = didx[sl]

            pltpu.async_copy(stage, acc.at[sdidx], sem=ssem, add=True)

        load_and_start(0, 0)
        _zero_tile_slice(stage1, acc, s, _D2, _C2)
        plsc.subcore_barrier()

        def pair(i, _):
            load_and_start(1, 2 * i + 1)
            wait_gathers(0)

            @pl.when(i > 0)
            def _():
                wait_scatter(0)

            compute_scatter(0)

            @pl.when(i + 1 < npair)
            def _():
                load_and_start(0, 2 * i + 2)

            wait_gathers(1)

            @pl.when(i > 0)
            def _():
                wait_scatter(1)

            compute_scatter(1)
            return 0

        lax.fori_loop(0, npair, pair, 0)
        wait_scatter(0)
        wait_scatter(1)
        plsc.subcore_barrier()
        _copy_out_tile(acc, out_h, c, s)

    return k(t2, adst2, src, dst)


def kernel(x, edge_index, cluster_id, cluster_index, W1, att_src1, att_dst1, b1,
           W2, att_src2, att_dst2, b2):
    src = edge_index[0].astype(jnp.int32)
    dst = edge_index[1].astype(jnp.int32)

    t1, adst_tab = _tc_pre1(x, W1, att_src1, att_dst1)
    acc = _sc_gat1(t1, adst_tab, src, dst)
    h1 = _tc_post1(acc, t1, adst_tab, b1)

    sel_id = cluster_id[cluster_index]
    sel_x = h1[cluster_index]
    cf = (sel_id.T @ sel_x) / sel_id.sum(0)[:, None]
    cidx = jnp.argmax(cluster_id, axis=1)

    W2a, W2b = W2[:128], W2[128:]
    cfA = cf @ W2a
    cfB = cf @ W2b
    x1b = cfB[cidx]
    x1a = cfA[cidx]
    t2, adst2_tab = _tc_pre2(h1, W2a, W2b, x1b, x1a,
                             att_src2.reshape(-1), att_dst2.reshape(-1))

    acc2 = _sc_gat2(t2.reshape(2 * N, _D2),
                    adst2_tab.reshape(2 * N, _LANE), src, dst)

    out = _tc_post2(acc2, t2, adst2_tab, b2)
    return out.reshape(2 * N, NCLASS * NCLASS)
